# Initial kernel scaffold; baseline (speedup 1.0000x reference)
import jax
import jax.numpy as jnp
from jax.experimental import pallas as pl


def kernel(mol_x, mol_edge_index, mol_batch, squence, W_gcn, b_gcn, W1, b1, W2, b2, W3, b3):
    raise NotImplementedError("write your pallas kernel here")


# speedup vs baseline: 12.6930x; 12.6930x over previous
import functools

import jax
import jax.numpy as jnp
from jax import lax
from jax.experimental import pallas as pl
from jax.experimental.pallas import tpu as pltpu
from jax.experimental.pallas import tpu_sc as plsc

N_NODES = 50000
N_EDGES = 800000
B = 512
D_MOL = 78
D_A = 64
D_B = 16
D_OUT = 128

NC, NS, L = 2, 16, 16
NW = NC * NS

NPAD = 50176
EPAD = 802816
HALF = NPAD // 2
BAG_ROWS = HALF + 512
DEG_ROWS = NPAD + 256
EPT_A = EPAD // NW // 128
EPT_C = EPAD // NS // 128
CBLK = 14
RPT = NPAD // NW
SEG_ROWS = 520


def _zero_vmem_2d(ref, rows, cols):
    def body(i, _):
        for g in range(cols // 16):
            ref[i, pl.ds(g * 16, 16)] = jnp.zeros((16,), jnp.float32)
        return 0
    lax.fori_loop(0, rows, body, 0)


def _deg_body(dst_hbm, deg_hbm, deg_s, dst_v, ones_v, zro_v):
    cid = lax.axis_index("c")
    sid = lax.axis_index("s")
    wid = cid * NS + sid
    for g in range(128 // 16):
        ones_v[pl.ds(g * 16, 16)] = jnp.ones((16,), jnp.float32)

    def zb(i, _):
        zro_v[pl.ds(i * 16, 16)] = jnp.zeros((16,), jnp.float32)
        return 0
    lax.fori_loop(0, DEG_ROWS // NS // 16, zb, 0)
    pltpu.sync_copy(zro_v, deg_s.at[pl.ds(sid * (DEG_ROWS // NS), DEG_ROWS // NS)])
    plsc.subcore_barrier()
    pltpu.sync_copy(dst_hbm.at[wid], dst_v)

    def step(j, _):
        pltpu.sync_copy(ones_v, deg_s.at[dst_v.at[j]], add=True)
        return 0
    lax.fori_loop(0, EPT_A, step, 0)
    plsc.subcore_barrier()
    st = NPAD // NS
    pltpu.sync_copy(deg_s.at[pl.ds(sid * st, st)], zro_v.at[pl.ds(0, st)])
    pltpu.sync_copy(zro_v.at[pl.ds(0, st)],
                    deg_hbm.at[pl.ds(cid * NPAD + sid * st, st)])


@functools.partial(
    pl.kernel,
    out_type=jax.ShapeDtypeStruct((NC * NPAD,), jnp.float32),
    mesh=plsc.VectorSubcoreMesh(core_axis_name="c", subcore_axis_name="s"),
    compiler_params=pltpu.CompilerParams(use_tc_tiling_on_sc=False),
    scratch_types=[
        pltpu.VMEM_SHARED((DEG_ROWS,), jnp.float32),
        pltpu.VMEM((EPT_A, 128), jnp.int32),
        pltpu.VMEM((128,), jnp.float32),
        pltpu.VMEM((DEG_ROWS // NS,), jnp.float32),
    ],
)
def _deg_kernel(dst_hbm, deg_hbm, deg_s, dst_v, ones_v, zro_v):
    _deg_body(dst_hbm, deg_hbm, deg_s, dst_v, ones_v, zro_v)


def _dis_body(d0_ref, d1_ref, dis_ref):
    deg = d0_ref[...] + d1_ref[...] + 1.0
    dis_ref[...] = lax.rsqrt(deg)


def _xs_body(x_ref, dis_ref, xsa_ref, xsb_ref):
    xsa_ref[...] = x_ref[:, :D_A] * dis_ref[...]
    xsb_ref[:, :D_MOL - D_A] = x_ref[:, D_A:] * dis_ref[...]
    xsb_ref[:, D_MOL - D_A:] = jnp.zeros(
        (x_ref.shape[0], D_B - (D_MOL - D_A)), jnp.float32)


def _make_bag_kernel(depth):
    def body(src_hbm, dst_hbm, xs_hbm, bag_hbm, bag_s, src_v, dst_v,
             rows_v, idx_v):
        cid = lax.axis_index("c")
        sid = lax.axis_index("s")
        base_node = cid * HALF
        _zero_vmem_2d(rows_v, 128, depth)
        zpt = BAG_ROWS // NS
        for k in range(zpt // 128):
            pltpu.sync_copy(rows_v,
                            bag_s.at[pl.ds(sid * zpt + k * 128, 128)])
        for k in range(zpt // 128 * 128, zpt, 64):
            pltpu.sync_copy(rows_v.at[pl.ds(0, 64)],
                            bag_s.at[pl.ds(sid * zpt + k, 64)])
        plsc.subcore_barrier()

        def block(sb, _):
            pltpu.sync_copy(src_hbm.at[sid, pl.ds(sb * CBLK, CBLK)], src_v)
            pltpu.sync_copy(dst_hbm.at[sid, pl.ds(sb * CBLK, CBLK)], dst_v)

            def step(j, _):
                pltpu.sync_copy(xs_hbm.at[src_v.at[j]], rows_v)
                for g in range(8):
                    v = dst_v[j, pl.ds(g * 16, 16)]
                    lv = v - base_node
                    ok = (lv >= 0) & (lv < HALF)
                    idx_v[pl.ds(g * 16, 16)] = jnp.where(ok, lv, HALF)
                pltpu.sync_copy(rows_v, bag_s.at[idx_v], add=True)
                return 0
            lax.fori_loop(0, CBLK, step, 0)
            return 0
        lax.fori_loop(0, EPT_C // CBLK, block, 0)
        plsc.subcore_barrier()
        st = HALF // NS
        off = 0
        while off < st:
            n = min(128, st - off)
            pltpu.sync_copy(bag_s.at[pl.ds(sid * st + off, n)],
                            rows_v.at[pl.ds(0, n)])
            pltpu.sync_copy(rows_v.at[pl.ds(0, n)],
                            bag_hbm.at[pl.ds(base_node + sid * st + off, n)])
            off += n

    return pl.kernel(
        body,
        out_type=jax.ShapeDtypeStruct((NPAD, depth), jnp.float32),
        mesh=plsc.VectorSubcoreMesh(core_axis_name="c", subcore_axis_name="s"),
        compiler_params=pltpu.CompilerParams(use_tc_tiling_on_sc=False),
        scratch_types=[
            pltpu.VMEM_SHARED((BAG_ROWS, depth), jnp.float32),
            pltpu.VMEM((CBLK, 128), jnp.int32),
            pltpu.VMEM((CBLK, 128), jnp.int32),
            pltpu.VMEM((128, depth), jnp.float32),
            pltpu.VMEM((128,), jnp.int32),
        ],
    )


_bag_kernel_a = _make_bag_kernel(D_A)
_bag_kernel_b = _make_bag_kernel(D_B)


def _h_body(baga_ref, xsa_ref, bagb_ref, xsb_ref, dis_ref, wa_ref, wb_ref,
            b_ref, h_ref):
    ta = dis_ref[...] * (baga_ref[...] + xsa_ref[...])
    tb = dis_ref[...] * (bagb_ref[...] + xsb_ref[...])
    h = (jnp.dot(ta, wa_ref[...], preferred_element_type=jnp.float32)
         + jnp.dot(tb, wb_ref[...], preferred_element_type=jnp.float32))
    h_ref[...] = jnp.maximum(h + b_ref[...], 0.0)


def _segmax_body(h_hbm, batch_hbm, parts_hbm, out_v, hv, bv):
    cid = lax.axis_index("c")
    sid = lax.axis_index("s")
    wid = cid * NS + sid
    base = wid * RPT
    neg = jnp.full((16,), -jnp.inf, jnp.float32)

    def init(i, _):
        for g in range(8):
            out_v[i, pl.ds(g * 16, 16)] = neg
        return 0
    lax.fori_loop(0, SEG_ROWS, init, 0)
    for k in range(RPT // 112):
        pltpu.sync_copy(h_hbm.at[pl.ds(base + k * 112, 112)], hv)
        pltpu.sync_copy(batch_hbm.at[pl.ds(base + k * 112, 112)],
                        bv.at[pl.ds(0, 112)])

        def row(r, _):
            seg = bv[pl.ds(r, 16)][0]
            for g in range(8):
                cur = out_v[seg, pl.ds(g * 16, 16)]
                hval = hv[r, pl.ds(g * 16, 16)]
                out_v[seg, pl.ds(g * 16, 16)] = jnp.maximum(cur, hval)
            return 0
        lax.fori_loop(0, 112, row, 0)
    pltpu.sync_copy(out_v, parts_hbm.at[wid])


@functools.partial(
    pl.kernel,
    out_type=jax.ShapeDtypeStruct((NW, SEG_ROWS, D_OUT), jnp.float32),
    mesh=plsc.VectorSubcoreMesh(core_axis_name="c", subcore_axis_name="s"),
    compiler_params=pltpu.CompilerParams(use_tc_tiling_on_sc=False),
    scratch_types=[
        pltpu.VMEM((SEG_ROWS, D_OUT), jnp.float32),
        pltpu.VMEM((112, D_OUT), jnp.float32),
        pltpu.VMEM((128,), jnp.int32),
    ],
)
def _segmax_kernel(h_hbm, batch_hbm, parts_hbm, out_v, hv, bv):
    _segmax_body(h_hbm, batch_hbm, parts_hbm, out_v, hv, bv)


def _head_body(parts_ref, sq_ref, w1_ref, b1_ref, w2_ref, b2_ref,
               w3_ref, b3_ref, y_ref):
    pooled = jnp.max(parts_ref[:, :B, :], axis=0)
    x2 = jnp.maximum(jnp.dot(sq_ref[...], w1_ref[...],
                             preferred_element_type=jnp.float32)
                     + b1_ref[...], 0.0)
    x2 = jnp.maximum(jnp.dot(x2, w2_ref[...],
                             preferred_element_type=jnp.float32)
                     + b2_ref[...], 0.0)
    x2 = jnp.dot(x2, w3_ref[...],
                 preferred_element_type=jnp.float32) + b3_ref[...]
    y = lax.dot_general(pooled, x2, (((1,), (1,)), ((), ())),
                        preferred_element_type=jnp.float32)
    y_ref[...] = jax.nn.sigmoid(y)


def kernel(mol_x, mol_edge_index, mol_batch, squence,
           W_gcn, b_gcn, W1, b1, W2, b2, W3, b3):
    i32 = jnp.int32
    src = mol_edge_index[0].astype(i32)
    dst = mol_edge_index[1].astype(i32)
    epad = EPAD - N_EDGES
    src_p = jnp.concatenate([src, jnp.zeros((epad,), i32)])
    dst_p = jnp.concatenate([dst, jnp.full((epad,), NPAD, i32)])
    x_p = jnp.pad(mol_x, ((0, NPAD - N_NODES), (0, 0)))
    batch_p = jnp.concatenate(
        [mol_batch.astype(i32), jnp.full((NPAD - N_NODES,), B, i32)])
    w_p = jnp.pad(W_gcn, ((0, D_A + D_B - D_MOL), (0, 0)))

    deg_parts = _deg_kernel(dst_p.reshape(NW, EPT_A, 128))

    nblk = NPAD // 1024
    dis_flat = pl.pallas_call(
        _dis_body,
        out_shape=jax.ShapeDtypeStruct((NPAD // 128, 128), jnp.float32),
        grid=(nblk,),
        in_specs=[pl.BlockSpec((8, 128), lambda i: (i, 0)),
                  pl.BlockSpec((8, 128), lambda i: (i, 0))],
        out_specs=pl.BlockSpec((8, 128), lambda i: (i, 0)),
    )(deg_parts[:NPAD].reshape(NPAD // 128, 128),
      deg_parts[NPAD:].reshape(NPAD // 128, 128))
    dis_col = dis_flat.reshape(NPAD, 1)

    xs_a, xs_b = pl.pallas_call(
        _xs_body,
        out_shape=(jax.ShapeDtypeStruct((NPAD, D_A), jnp.float32),
                   jax.ShapeDtypeStruct((NPAD, D_B), jnp.float32)),
        grid=(nblk,),
        in_specs=[pl.BlockSpec((1024, D_MOL), lambda i: (i, 0)),
                  pl.BlockSpec((1024, 1), lambda i: (i, 0))],
        out_specs=(pl.BlockSpec((1024, D_A), lambda i: (i, 0)),
                   pl.BlockSpec((1024, D_B), lambda i: (i, 0))),
    )(x_p, dis_col)

    src_r = src_p.reshape(NS, EPT_C, 128)
    dst_r = dst_p.reshape(NS, EPT_C, 128)
    bag_a = _bag_kernel_a(src_r, dst_r, xs_a)
    bag_b = _bag_kernel_b(src_r, dst_r, xs_b)

    h = pl.pallas_call(
        _h_body,
        out_shape=jax.ShapeDtypeStruct((NPAD, D_OUT), jnp.float32),
        grid=(nblk,),
        in_specs=[pl.BlockSpec((1024, D_A), lambda i: (i, 0)),
                  pl.BlockSpec((1024, D_A), lambda i: (i, 0)),
                  pl.BlockSpec((1024, D_B), lambda i: (i, 0)),
                  pl.BlockSpec((1024, D_B), lambda i: (i, 0)),
                  pl.BlockSpec((1024, 1), lambda i: (i, 0)),
                  pl.BlockSpec((D_A, D_OUT), lambda i: (0, 0)),
                  pl.BlockSpec((D_B, D_OUT), lambda i: (0, 0)),
                  pl.BlockSpec((1, D_OUT), lambda i: (0, 0))],
        out_specs=pl.BlockSpec((1024, D_OUT), lambda i: (i, 0)),
    )(bag_a, xs_a, bag_b, xs_b, dis_col, w_p[:D_A], w_p[D_A:],
      b_gcn.reshape(1, D_OUT))

    parts = _segmax_kernel(h, batch_p)

    y = pl.pallas_call(
        _head_body,
        out_shape=jax.ShapeDtypeStruct((B, B), jnp.float32),
    )(parts, squence, W1, b1.reshape(1, -1), W2, b2.reshape(1, -1),
      W3, b3.reshape(1, -1))
    return y


# speedup vs baseline: 13.0880x; 1.0311x over previous
import functools

import jax
import jax.numpy as jnp
from jax import lax
from jax.experimental import pallas as pl
from jax.experimental.pallas import tpu as pltpu
from jax.experimental.pallas import tpu_sc as plsc

N_NODES = 50000
N_EDGES = 800000
B = 512
D_MOL = 78
D_A = 64
D_B = 16
D_OUT = 128

NC, NS, L = 2, 16, 16
NW = NC * NS

NPAD = 50176
EPAD = 802816
HALF = NPAD // 2
BAG_ROWS = HALF + 512
DEG_ROWS = NPAD + 256
EPT_A = EPAD // NW // 128
EPT_C = EPAD // NS // 128
CBLK = 14
RPT = NPAD // NW
SEG_ROWS = 520


def _zero_vmem_2d(ref, rows, cols):
    def body(i, _):
        for g in range(cols // 16):
            ref[i, pl.ds(g * 16, 16)] = jnp.zeros((16,), jnp.float32)
        return 0
    lax.fori_loop(0, rows, body, 0)


def _deg_body(dst_hbm, deg_hbm, deg_s, dst_v, ones_v, zro_v):
    cid = lax.axis_index("c")
    sid = lax.axis_index("s")
    wid = cid * NS + sid
    for g in range(128 // 16):
        ones_v[pl.ds(g * 16, 16)] = jnp.ones((16,), jnp.float32)

    def zb(i, _):
        zro_v[pl.ds(i * 16, 16)] = jnp.zeros((16,), jnp.float32)
        return 0
    lax.fori_loop(0, DEG_ROWS // NS // 16, zb, 0)
    pltpu.sync_copy(zro_v, deg_s.at[pl.ds(sid * (DEG_ROWS // NS), DEG_ROWS // NS)])
    plsc.subcore_barrier()
    pltpu.sync_copy(dst_hbm.at[wid], dst_v)

    def step(j, _):
        pltpu.sync_copy(ones_v, deg_s.at[dst_v.at[j]], add=True)
        return 0
    lax.fori_loop(0, EPT_A, step, 0)
    plsc.subcore_barrier()
    st = NPAD // NS
    pltpu.sync_copy(deg_s.at[pl.ds(sid * st, st)], zro_v.at[pl.ds(0, st)])
    pltpu.sync_copy(zro_v.at[pl.ds(0, st)],
                    deg_hbm.at[pl.ds(cid * NPAD + sid * st, st)])


@functools.partial(
    pl.kernel,
    out_type=jax.ShapeDtypeStruct((NC * NPAD,), jnp.float32),
    mesh=plsc.VectorSubcoreMesh(core_axis_name="c", subcore_axis_name="s"),
    compiler_params=pltpu.CompilerParams(use_tc_tiling_on_sc=False),
    scratch_types=[
        pltpu.VMEM_SHARED((DEG_ROWS,), jnp.float32),
        pltpu.VMEM((EPT_A, 128), jnp.int32),
        pltpu.VMEM((128,), jnp.float32),
        pltpu.VMEM((DEG_ROWS // NS,), jnp.float32),
    ],
)
def _deg_kernel(dst_hbm, deg_hbm, deg_s, dst_v, ones_v, zro_v):
    _deg_body(dst_hbm, deg_hbm, deg_s, dst_v, ones_v, zro_v)


def _dis_body(d0_ref, d1_ref, dis_ref):
    deg = d0_ref[...] + d1_ref[...] + 1.0
    dis_ref[...] = lax.rsqrt(deg)


def _xs_body(x_ref, dis_ref, xsa_ref, xsb_ref):
    xsa_ref[...] = x_ref[:, :D_A] * dis_ref[...]
    xsb_ref[:, :D_MOL - D_A] = x_ref[:, D_A:] * dis_ref[...]
    xsb_ref[:, D_MOL - D_A:] = jnp.zeros(
        (x_ref.shape[0], D_B - (D_MOL - D_A)), jnp.float32)


NBLK = EPT_C // CBLK


def _make_bag_kernel(depth):
    def body(src_hbm, dst_hbm, xs_hbm, bag_hbm, bag_s, src_v, dst_v,
             rows0, rows1, idx0, idx1, sg0, sg1, ss0, ss1):
        cid = lax.axis_index("c")
        sid = lax.axis_index("s")
        base_node = cid * HALF
        rows = (rows0, rows1)
        idxs = (idx0, idx1)
        sgs = (sg0, sg1)
        sss = (ss0, ss1)
        _zero_vmem_2d(rows0, 128, depth)
        zpt = BAG_ROWS // NS
        for k in range(zpt // 128):
            pltpu.sync_copy(rows0,
                            bag_s.at[pl.ds(sid * zpt + k * 128, 128)])
        for k in range(zpt // 128 * 128, zpt, 64):
            pltpu.sync_copy(rows0.at[pl.ds(0, 64)],
                            bag_s.at[pl.ds(sid * zpt + k, 64)])
        plsc.subcore_barrier()

        pltpu.sync_copy(src_hbm.at[sid, pl.ds(0, CBLK)], src_v)
        pltpu.sync_copy(dst_hbm.at[sid, pl.ds(0, CBLK)], dst_v)
        pltpu.async_copy(xs_hbm.at[src_v.at[0]], rows0, sg0)

        def block(sb, _):
            gather_d = None
            scat_d = None
            for jj in range(CBLK):
                s = jj % 2
                o = 1 - s
                if gather_d is None:
                    pltpu.make_async_copy(
                        xs_hbm.at[src_v.at[0]], rows0, sg0).wait()
                else:
                    gather_d.wait()
                for g in range(8):
                    v = dst_v[jj, pl.ds(g * 16, 16)]
                    lv = v - base_node
                    ok = (lv >= 0) & (lv < HALF)
                    idxs[s][pl.ds(g * 16, 16)] = jnp.where(ok, lv, HALF)
                new_scat = pltpu.async_copy(rows[s], bag_s.at[idxs[s]],
                                            sss[s], add=True)
                if scat_d is None:
                    @pl.when(sb > 0)
                    def _():
                        pltpu.make_async_copy(
                            rows1, bag_s.at[idx1], ss1).wait()
                else:
                    scat_d.wait()
                scat_d = new_scat
                if jj + 1 < CBLK:
                    gather_d = pltpu.async_copy(
                        xs_hbm.at[src_v.at[jj + 1]], rows[o], sgs[o])
                else:
                    gather_d = None

                    @pl.when(sb < NBLK - 1)
                    def _():
                        pltpu.sync_copy(
                            src_hbm.at[sid, pl.ds((sb + 1) * CBLK, CBLK)],
                            src_v)
                        pltpu.sync_copy(
                            dst_hbm.at[sid, pl.ds((sb + 1) * CBLK, CBLK)],
                            dst_v)
                        pltpu.async_copy(xs_hbm.at[src_v.at[0]], rows0, sg0)
            return 0
        lax.fori_loop(0, NBLK, block, 0)
        pltpu.make_async_copy(rows1, bag_s.at[idx1], ss1).wait()
        plsc.subcore_barrier()
        st = HALF // NS
        off = 0
        while off < st:
            n = min(128, st - off)
            pltpu.sync_copy(bag_s.at[pl.ds(sid * st + off, n)],
                            rows0.at[pl.ds(0, n)])
            pltpu.sync_copy(rows0.at[pl.ds(0, n)],
                            bag_hbm.at[pl.ds(base_node + sid * st + off, n)])
            off += n

    return pl.kernel(
        body,
        out_type=jax.ShapeDtypeStruct((NPAD, depth), jnp.float32),
        mesh=plsc.VectorSubcoreMesh(core_axis_name="c", subcore_axis_name="s"),
        compiler_params=pltpu.CompilerParams(use_tc_tiling_on_sc=False),
        scratch_types=[
            pltpu.VMEM_SHARED((BAG_ROWS, depth), jnp.float32),
            pltpu.VMEM((CBLK, 128), jnp.int32),
            pltpu.VMEM((CBLK, 128), jnp.int32),
            pltpu.VMEM((128, depth), jnp.float32),
            pltpu.VMEM((128, depth), jnp.float32),
            pltpu.VMEM((128,), jnp.int32),
            pltpu.VMEM((128,), jnp.int32),
            pltpu.SemaphoreType.DMA,
            pltpu.SemaphoreType.DMA,
            pltpu.SemaphoreType.DMA,
            pltpu.SemaphoreType.DMA,
        ],
    )


_bag_kernel_a = _make_bag_kernel(D_A)
_bag_kernel_b = _make_bag_kernel(D_B)


def _h_body(baga_ref, xsa_ref, bagb_ref, xsb_ref, dis_ref, wa_ref, wb_ref,
            b_ref, h_ref):
    ta = dis_ref[...] * (baga_ref[...] + xsa_ref[...])
    tb = dis_ref[...] * (bagb_ref[...] + xsb_ref[...])
    h = (jnp.dot(ta, wa_ref[...], preferred_element_type=jnp.float32)
         + jnp.dot(tb, wb_ref[...], preferred_element_type=jnp.float32))
    h_ref[...] = jnp.maximum(h + b_ref[...], 0.0)


def _segmax_body(h_hbm, batch_hbm, parts_hbm, out_v, hv, bv):
    cid = lax.axis_index("c")
    sid = lax.axis_index("s")
    wid = cid * NS + sid
    base = wid * RPT
    neg = jnp.full((16,), -jnp.inf, jnp.float32)

    def init(i, _):
        for g in range(8):
            out_v[i, pl.ds(g * 16, 16)] = neg
        return 0
    lax.fori_loop(0, SEG_ROWS, init, 0)
    for k in range(RPT // 112):
        pltpu.sync_copy(h_hbm.at[pl.ds(base + k * 112, 112)], hv)
        pltpu.sync_copy(batch_hbm.at[pl.ds(base + k * 112, 112)],
                        bv.at[pl.ds(0, 112)])

        def row(r, _):
            seg = bv[pl.ds(r, 16)][0]
            for g in range(8):
                cur = out_v[seg, pl.ds(g * 16, 16)]
                hval = hv[r, pl.ds(g * 16, 16)]
                out_v[seg, pl.ds(g * 16, 16)] = jnp.maximum(cur, hval)
            return 0
        lax.fori_loop(0, 112, row, 0)
    pltpu.sync_copy(out_v, parts_hbm.at[wid])


@functools.partial(
    pl.kernel,
    out_type=jax.ShapeDtypeStruct((NW, SEG_ROWS, D_OUT), jnp.float32),
    mesh=plsc.VectorSubcoreMesh(core_axis_name="c", subcore_axis_name="s"),
    compiler_params=pltpu.CompilerParams(use_tc_tiling_on_sc=False),
    scratch_types=[
        pltpu.VMEM((SEG_ROWS, D_OUT), jnp.float32),
        pltpu.VMEM((112, D_OUT), jnp.float32),
        pltpu.VMEM((128,), jnp.int32),
    ],
)
def _segmax_kernel(h_hbm, batch_hbm, parts_hbm, out_v, hv, bv):
    _segmax_body(h_hbm, batch_hbm, parts_hbm, out_v, hv, bv)


def _head_body(parts_ref, sq_ref, w1_ref, b1_ref, w2_ref, b2_ref,
               w3_ref, b3_ref, y_ref):
    pooled = jnp.max(parts_ref[:, :B, :], axis=0)
    x2 = jnp.maximum(jnp.dot(sq_ref[...], w1_ref[...],
                             preferred_element_type=jnp.float32)
                     + b1_ref[...], 0.0)
    x2 = jnp.maximum(jnp.dot(x2, w2_ref[...],
                             preferred_element_type=jnp.float32)
                     + b2_ref[...], 0.0)
    x2 = jnp.dot(x2, w3_ref[...],
                 preferred_element_type=jnp.float32) + b3_ref[...]
    y = lax.dot_general(pooled, x2, (((1,), (1,)), ((), ())),
                        preferred_element_type=jnp.float32)
    y_ref[...] = jax.nn.sigmoid(y)


def kernel(mol_x, mol_edge_index, mol_batch, squence,
           W_gcn, b_gcn, W1, b1, W2, b2, W3, b3):
    i32 = jnp.int32
    src = mol_edge_index[0].astype(i32)
    dst = mol_edge_index[1].astype(i32)
    epad = EPAD - N_EDGES
    src_p = jnp.concatenate([src, jnp.zeros((epad,), i32)])
    dst_p = jnp.concatenate([dst, jnp.full((epad,), NPAD, i32)])
    x_p = jnp.pad(mol_x, ((0, NPAD - N_NODES), (0, 0)))
    batch_p = jnp.concatenate(
        [mol_batch.astype(i32), jnp.full((NPAD - N_NODES,), B, i32)])
    w_p = jnp.pad(W_gcn, ((0, D_A + D_B - D_MOL), (0, 0)))

    deg_parts = _deg_kernel(dst_p.reshape(NW, EPT_A, 128))

    nblk = NPAD // 1024
    dis_flat = pl.pallas_call(
        _dis_body,
        out_shape=jax.ShapeDtypeStruct((NPAD // 128, 128), jnp.float32),
        grid=(nblk,),
        in_specs=[pl.BlockSpec((8, 128), lambda i: (i, 0)),
                  pl.BlockSpec((8, 128), lambda i: (i, 0))],
        out_specs=pl.BlockSpec((8, 128), lambda i: (i, 0)),
    )(deg_parts[:NPAD].reshape(NPAD // 128, 128),
      deg_parts[NPAD:].reshape(NPAD // 128, 128))
    dis_col = dis_flat.reshape(NPAD, 1)

    xs_a, xs_b = pl.pallas_call(
        _xs_body,
        out_shape=(jax.ShapeDtypeStruct((NPAD, D_A), jnp.float32),
                   jax.ShapeDtypeStruct((NPAD, D_B), jnp.float32)),
        grid=(nblk,),
        in_specs=[pl.BlockSpec((1024, D_MOL), lambda i: (i, 0)),
                  pl.BlockSpec((1024, 1), lambda i: (i, 0))],
        out_specs=(pl.BlockSpec((1024, D_A), lambda i: (i, 0)),
                   pl.BlockSpec((1024, D_B), lambda i: (i, 0))),
    )(x_p, dis_col)

    src_r = src_p.reshape(NS, EPT_C, 128)
    dst_r = dst_p.reshape(NS, EPT_C, 128)
    bag_a = _bag_kernel_a(src_r, dst_r, xs_a)
    bag_b = _bag_kernel_b(src_r, dst_r, xs_b)

    h = pl.pallas_call(
        _h_body,
        out_shape=jax.ShapeDtypeStruct((NPAD, D_OUT), jnp.float32),
        grid=(nblk,),
        in_specs=[pl.BlockSpec((1024, D_A), lambda i: (i, 0)),
                  pl.BlockSpec((1024, D_A), lambda i: (i, 0)),
                  pl.BlockSpec((1024, D_B), lambda i: (i, 0)),
                  pl.BlockSpec((1024, D_B), lambda i: (i, 0)),
                  pl.BlockSpec((1024, 1), lambda i: (i, 0)),
                  pl.BlockSpec((D_A, D_OUT), lambda i: (0, 0)),
                  pl.BlockSpec((D_B, D_OUT), lambda i: (0, 0)),
                  pl.BlockSpec((1, D_OUT), lambda i: (0, 0))],
        out_specs=pl.BlockSpec((1024, D_OUT), lambda i: (i, 0)),
    )(bag_a, xs_a, bag_b, xs_b, dis_col, w_p[:D_A], w_p[D_A:],
      b_gcn.reshape(1, D_OUT))

    parts = _segmax_kernel(h, batch_p)

    y = pl.pallas_call(
        _head_body,
        out_shape=jax.ShapeDtypeStruct((B, B), jnp.float32),
    )(parts, squence, W1, b1.reshape(1, -1), W2, b2.reshape(1, -1),
      W3, b3.reshape(1, -1))
    return y


# speedup vs baseline: 13.2854x; 1.0151x over previous
import functools

import jax
import jax.numpy as jnp
from jax import lax
from jax.experimental import pallas as pl
from jax.experimental.pallas import tpu as pltpu
from jax.experimental.pallas import tpu_sc as plsc

N_NODES = 50000
N_EDGES = 800000
B = 512
D_MOL = 78
D_A = 64
D_B = 16
D_OUT = 128

NC, NS, L = 2, 16, 16
NW = NC * NS

NPAD = 50176
EPAD = 802816
HALF = NPAD // 2
BAG_ROWS = HALF + 512
DEG_ROWS = NPAD + 256
EPT_A = EPAD // NW // 128
EPT_C = EPAD // NS // 128
CBLK = 14
RPT = NPAD // NW
SEG_ROWS = 520


def _zero_vmem_2d(ref, rows, cols):
    def body(i, _):
        for g in range(cols // 16):
            ref[i, pl.ds(g * 16, 16)] = jnp.zeros((16,), jnp.float32)
        return 0
    lax.fori_loop(0, rows, body, 0)


def _deg_body(dst_hbm, deg_hbm, deg_s, dst_v, ones_v, zro_v):
    cid = lax.axis_index("c")
    sid = lax.axis_index("s")
    wid = cid * NS + sid
    for g in range(128 // 16):
        ones_v[pl.ds(g * 16, 16)] = jnp.ones((16,), jnp.float32)

    def zb(i, _):
        zro_v[pl.ds(i * 16, 16)] = jnp.zeros((16,), jnp.float32)
        return 0
    lax.fori_loop(0, DEG_ROWS // NS // 16, zb, 0)
    pltpu.sync_copy(zro_v, deg_s.at[pl.ds(sid * (DEG_ROWS // NS), DEG_ROWS // NS)])
    plsc.subcore_barrier()
    pltpu.sync_copy(dst_hbm.at[wid], dst_v)

    def step(j, _):
        pltpu.sync_copy(ones_v, deg_s.at[dst_v.at[j]], add=True)
        return 0
    lax.fori_loop(0, EPT_A, step, 0)
    plsc.subcore_barrier()
    st = NPAD // NS
    pltpu.sync_copy(deg_s.at[pl.ds(sid * st, st)], zro_v.at[pl.ds(0, st)])
    pltpu.sync_copy(zro_v.at[pl.ds(0, st)],
                    deg_hbm.at[pl.ds(cid * NPAD + sid * st, st)])


@functools.partial(
    pl.kernel,
    out_type=jax.ShapeDtypeStruct((NC * NPAD,), jnp.float32),
    mesh=plsc.VectorSubcoreMesh(core_axis_name="c", subcore_axis_name="s"),
    compiler_params=pltpu.CompilerParams(use_tc_tiling_on_sc=False),
    scratch_types=[
        pltpu.VMEM_SHARED((DEG_ROWS,), jnp.float32),
        pltpu.VMEM((EPT_A, 128), jnp.int32),
        pltpu.VMEM((128,), jnp.float32),
        pltpu.VMEM((DEG_ROWS // NS,), jnp.float32),
    ],
)
def _deg_kernel(dst_hbm, deg_hbm, deg_s, dst_v, ones_v, zro_v):
    _deg_body(dst_hbm, deg_hbm, deg_s, dst_v, ones_v, zro_v)


def _dis_body(d0_ref, d1_ref, dis_ref):
    deg = d0_ref[...] + d1_ref[...] + 1.0
    dis_ref[...] = lax.rsqrt(deg)


def _xs_body(x_ref, dis_ref, xsa_ref, xsb_ref):
    xsa_ref[...] = x_ref[:, :D_A] * dis_ref[...]
    xsb_ref[:, :D_MOL - D_A] = x_ref[:, D_A:] * dis_ref[...]
    xsb_ref[:, D_MOL - D_A:] = jnp.zeros(
        (x_ref.shape[0], D_B - (D_MOL - D_A)), jnp.float32)


NBLK = EPT_C // CBLK


def _make_bag_kernel(depth):
    def body(src_hbm, dst_hbm, xs_hbm, bag_hbm, bag_s, src_v, dst_v,
             rows0, rows1, idx0, idx1, sg0, sg1, ss0, ss1):
        cid = lax.axis_index("c")
        sid = lax.axis_index("s")
        base_node = cid * HALF
        rows = (rows0, rows1)
        idxs = (idx0, idx1)
        sgs = (sg0, sg1)
        sss = (ss0, ss1)
        _zero_vmem_2d(rows0, 128, depth)
        zpt = BAG_ROWS // NS
        for k in range(zpt // 128):
            pltpu.sync_copy(rows0,
                            bag_s.at[pl.ds(sid * zpt + k * 128, 128)])
        for k in range(zpt // 128 * 128, zpt, 64):
            pltpu.sync_copy(rows0.at[pl.ds(0, 64)],
                            bag_s.at[pl.ds(sid * zpt + k, 64)])
        plsc.subcore_barrier()

        pltpu.sync_copy(src_hbm.at[sid, pl.ds(0, CBLK)], src_v)
        pltpu.sync_copy(dst_hbm.at[sid, pl.ds(0, CBLK)], dst_v)
        pltpu.async_copy(xs_hbm.at[src_v.at[0]], rows0, sg0)

        def block(sb, _):
            gather_d = None
            scat_d = None
            for jj in range(CBLK):
                s = jj % 2
                o = 1 - s
                if gather_d is None:
                    pltpu.make_async_copy(
                        xs_hbm.at[src_v.at[0]], rows0, sg0).wait()
                else:
                    gather_d.wait()
                for g in range(8):
                    v = dst_v[jj, pl.ds(g * 16, 16)]
                    lv = v - base_node
                    ok = (lv >= 0) & (lv < HALF)
                    idxs[s][pl.ds(g * 16, 16)] = jnp.where(ok, lv, HALF)
                new_scat = pltpu.async_copy(rows[s], bag_s.at[idxs[s]],
                                            sss[s], add=True)
                if scat_d is None:
                    @pl.when(sb > 0)
                    def _():
                        pltpu.make_async_copy(
                            rows1, bag_s.at[idx1], ss1).wait()
                else:
                    scat_d.wait()
                scat_d = new_scat
                if jj + 1 < CBLK:
                    gather_d = pltpu.async_copy(
                        xs_hbm.at[src_v.at[jj + 1]], rows[o], sgs[o])
                else:
                    gather_d = None

                    @pl.when(sb < NBLK - 1)
                    def _():
                        pltpu.sync_copy(
                            src_hbm.at[sid, pl.ds((sb + 1) * CBLK, CBLK)],
                            src_v)
                        pltpu.sync_copy(
                            dst_hbm.at[sid, pl.ds((sb + 1) * CBLK, CBLK)],
                            dst_v)
                        pltpu.async_copy(xs_hbm.at[src_v.at[0]], rows0, sg0)
            return 0
        lax.fori_loop(0, NBLK, block, 0)
        pltpu.make_async_copy(rows1, bag_s.at[idx1], ss1).wait()
        plsc.subcore_barrier()
        st = HALF // NS
        off = 0
        while off < st:
            n = min(128, st - off)
            pltpu.sync_copy(bag_s.at[pl.ds(sid * st + off, n)],
                            rows0.at[pl.ds(0, n)])
            pltpu.sync_copy(rows0.at[pl.ds(0, n)],
                            bag_hbm.at[pl.ds(base_node + sid * st + off, n)])
            off += n

    return pl.kernel(
        body,
        out_type=jax.ShapeDtypeStruct((NPAD, depth), jnp.float32),
        mesh=plsc.VectorSubcoreMesh(core_axis_name="c", subcore_axis_name="s"),
        compiler_params=pltpu.CompilerParams(use_tc_tiling_on_sc=False),
        scratch_types=[
            pltpu.VMEM_SHARED((BAG_ROWS, depth), jnp.float32),
            pltpu.VMEM((CBLK, 128), jnp.int32),
            pltpu.VMEM((CBLK, 128), jnp.int32),
            pltpu.VMEM((128, depth), jnp.float32),
            pltpu.VMEM((128, depth), jnp.float32),
            pltpu.VMEM((128,), jnp.int32),
            pltpu.VMEM((128,), jnp.int32),
            pltpu.SemaphoreType.DMA,
            pltpu.SemaphoreType.DMA,
            pltpu.SemaphoreType.DMA,
            pltpu.SemaphoreType.DMA,
        ],
    )


_bag_kernel_a = _make_bag_kernel(D_A)
_bag_kernel_b = _make_bag_kernel(D_B)


def _h_body(baga_ref, xsa_ref, bagb_ref, xsb_ref, dis_ref, wa_ref, wb_ref,
            b_ref, h_ref):
    ta = dis_ref[...] * (baga_ref[...] + xsa_ref[...])
    tb = dis_ref[...] * (bagb_ref[...] + xsb_ref[...])
    h = (jnp.dot(ta, wa_ref[...], preferred_element_type=jnp.float32)
         + jnp.dot(tb, wb_ref[...], preferred_element_type=jnp.float32))
    h_ref[...] = jnp.maximum(h + b_ref[...], 0.0)


def _segmax_body(h_hbm, batch_hbm, parts_hbm, out_v, hv0, hv1, bv0, bv1,
                 sh0, sh1, sb0, sb1):
    cid = lax.axis_index("c")
    sid = lax.axis_index("s")
    wid = cid * NS + sid
    base = wid * RPT
    neg = jnp.full((16,), -jnp.inf, jnp.float32)
    hvs = (hv0, hv1)
    bvs = (bv0, bv1)
    shs = (sh0, sh1)
    sbs = (sb0, sb1)

    def init(i, _):
        for g in range(8):
            out_v[i, pl.ds(g * 16, 16)] = neg
        return 0
    lax.fori_loop(0, SEG_ROWS, init, 0)
    nchunk = RPT // 112
    hd = pltpu.async_copy(h_hbm.at[pl.ds(base, 112)], hv0, sh0)
    bd = pltpu.async_copy(batch_hbm.at[pl.ds(base, 112)],
                          bv0.at[pl.ds(0, 112)], sb0)
    descs = (hd, bd)
    for k in range(nchunk):
        p = k % 2
        o = 1 - p
        descs[0].wait()
        descs[1].wait()
        if k + 1 < nchunk:
            off = base + (k + 1) * 112
            hd = pltpu.async_copy(h_hbm.at[pl.ds(off, 112)], hvs[o], shs[o])
            bd = pltpu.async_copy(batch_hbm.at[pl.ds(off, 112)],
                                  bvs[o].at[pl.ds(0, 112)], sbs[o])
            descs = (hd, bd)
        hv = hvs[p]
        bv = bvs[p]

        def row(r, _):
            seg = bv[pl.ds(r, 16)][0]
            for g in range(8):
                cur = out_v[seg, pl.ds(g * 16, 16)]
                hval = hv[r, pl.ds(g * 16, 16)]
                out_v[seg, pl.ds(g * 16, 16)] = jnp.maximum(cur, hval)
            return 0
        lax.fori_loop(0, 112, row, 0)
    pltpu.sync_copy(out_v, parts_hbm.at[wid])


@functools.partial(
    pl.kernel,
    out_type=jax.ShapeDtypeStruct((NW, SEG_ROWS, D_OUT), jnp.float32),
    mesh=plsc.VectorSubcoreMesh(core_axis_name="c", subcore_axis_name="s"),
    compiler_params=pltpu.CompilerParams(use_tc_tiling_on_sc=False),
    scratch_types=[
        pltpu.VMEM((SEG_ROWS, D_OUT), jnp.float32),
        pltpu.VMEM((112, D_OUT), jnp.float32),
        pltpu.VMEM((112, D_OUT), jnp.float32),
        pltpu.VMEM((128,), jnp.int32),
        pltpu.VMEM((128,), jnp.int32),
        pltpu.SemaphoreType.DMA,
        pltpu.SemaphoreType.DMA,
        pltpu.SemaphoreType.DMA,
        pltpu.SemaphoreType.DMA,
    ],
)
def _segmax_kernel(h_hbm, batch_hbm, parts_hbm, out_v, hv0, hv1, bv0, bv1,
                   sh0, sh1, sb0, sb1):
    _segmax_body(h_hbm, batch_hbm, parts_hbm, out_v, hv0, hv1, bv0, bv1,
                 sh0, sh1, sb0, sb1)


def _head_body(parts_ref, sq_ref, w1_ref, b1_ref, w2_ref, b2_ref,
               w3_ref, b3_ref, y_ref):
    pooled = jnp.max(parts_ref[:, :B, :], axis=0)
    x2 = jnp.maximum(jnp.dot(sq_ref[...], w1_ref[...],
                             preferred_element_type=jnp.float32)
                     + b1_ref[...], 0.0)
    x2 = jnp.maximum(jnp.dot(x2, w2_ref[...],
                             preferred_element_type=jnp.float32)
                     + b2_ref[...], 0.0)
    x2 = jnp.dot(x2, w3_ref[...],
                 preferred_element_type=jnp.float32) + b3_ref[...]
    y = lax.dot_general(pooled, x2, (((1,), (1,)), ((), ())),
                        preferred_element_type=jnp.float32)
    y_ref[...] = jax.nn.sigmoid(y)


def kernel(mol_x, mol_edge_index, mol_batch, squence,
           W_gcn, b_gcn, W1, b1, W2, b2, W3, b3):
    i32 = jnp.int32
    src = mol_edge_index[0].astype(i32)
    dst = mol_edge_index[1].astype(i32)
    epad = EPAD - N_EDGES
    src_p = jnp.concatenate([src, jnp.zeros((epad,), i32)])
    dst_p = jnp.concatenate([dst, jnp.full((epad,), NPAD, i32)])
    x_p = jnp.pad(mol_x, ((0, NPAD - N_NODES), (0, 0)))
    batch_p = jnp.concatenate(
        [mol_batch.astype(i32), jnp.full((NPAD - N_NODES,), B, i32)])
    w_p = jnp.pad(W_gcn, ((0, D_A + D_B - D_MOL), (0, 0)))

    deg_parts = _deg_kernel(dst_p.reshape(NW, EPT_A, 128))

    nblk = NPAD // 1024
    dis_flat = pl.pallas_call(
        _dis_body,
        out_shape=jax.ShapeDtypeStruct((NPAD // 128, 128), jnp.float32),
        grid=(nblk,),
        in_specs=[pl.BlockSpec((8, 128), lambda i: (i, 0)),
                  pl.BlockSpec((8, 128), lambda i: (i, 0))],
        out_specs=pl.BlockSpec((8, 128), lambda i: (i, 0)),
    )(deg_parts[:NPAD].reshape(NPAD // 128, 128),
      deg_parts[NPAD:].reshape(NPAD // 128, 128))
    dis_col = dis_flat.reshape(NPAD, 1)

    xs_a, xs_b = pl.pallas_call(
        _xs_body,
        out_shape=(jax.ShapeDtypeStruct((NPAD, D_A), jnp.float32),
                   jax.ShapeDtypeStruct((NPAD, D_B), jnp.float32)),
        grid=(nblk,),
        in_specs=[pl.BlockSpec((1024, D_MOL), lambda i: (i, 0)),
                  pl.BlockSpec((1024, 1), lambda i: (i, 0))],
        out_specs=(pl.BlockSpec((1024, D_A), lambda i: (i, 0)),
                   pl.BlockSpec((1024, D_B), lambda i: (i, 0))),
    )(x_p, dis_col)

    src_r = src_p.reshape(NS, EPT_C, 128)
    dst_r = dst_p.reshape(NS, EPT_C, 128)
    bag_a = _bag_kernel_a(src_r, dst_r, xs_a)
    bag_b = _bag_kernel_b(src_r, dst_r, xs_b)

    h = pl.pallas_call(
        _h_body,
        out_shape=jax.ShapeDtypeStruct((NPAD, D_OUT), jnp.float32),
        grid=(nblk,),
        in_specs=[pl.BlockSpec((1024, D_A), lambda i: (i, 0)),
                  pl.BlockSpec((1024, D_A), lambda i: (i, 0)),
                  pl.BlockSpec((1024, D_B), lambda i: (i, 0)),
                  pl.BlockSpec((1024, D_B), lambda i: (i, 0)),
                  pl.BlockSpec((1024, 1), lambda i: (i, 0)),
                  pl.BlockSpec((D_A, D_OUT), lambda i: (0, 0)),
                  pl.BlockSpec((D_B, D_OUT), lambda i: (0, 0)),
                  pl.BlockSpec((1, D_OUT), lambda i: (0, 0))],
        out_specs=pl.BlockSpec((1024, D_OUT), lambda i: (i, 0)),
    )(bag_a, xs_a, bag_b, xs_b, dis_col, w_p[:D_A], w_p[D_A:],
      b_gcn.reshape(1, D_OUT))

    parts = _segmax_kernel(h, batch_p)

    y = pl.pallas_call(
        _head_body,
        out_shape=jax.ShapeDtypeStruct((B, B), jnp.float32),
    )(parts, squence, W1, b1.reshape(1, -1), W2, b2.reshape(1, -1),
      W3, b3.reshape(1, -1))
    return y


# speedup vs baseline: 13.3310x; 1.0034x over previous
import functools

import jax
import jax.numpy as jnp
from jax import lax
from jax.experimental import pallas as pl
from jax.experimental.pallas import tpu as pltpu
from jax.experimental.pallas import tpu_sc as plsc

N_NODES = 50000
N_EDGES = 800000
B = 512
D_MOL = 78
D_A = 48
D_B = 32
D_OUT = 128

NC, NS, L = 2, 16, 16
NW = NC * NS

NPAD = 50176
EPAD = 802816
HALF = NPAD // 2
BAG_ROWS = HALF + 512
DEG_ROWS = NPAD + 256
EPT_A = EPAD // NW // 128
EPT_C = EPAD // NS // 128
CBLK = 14
RPT = NPAD // NW
SEG_ROWS = 520


def _zero_vmem_2d(ref, rows, cols):
    def body(i, _):
        for g in range(cols // 16):
            ref[i, pl.ds(g * 16, 16)] = jnp.zeros((16,), jnp.float32)
        return 0
    lax.fori_loop(0, rows, body, 0)


def _deg_body(dst_hbm, deg_hbm, deg_s, dst_v, ones_v, zro_v):
    cid = lax.axis_index("c")
    sid = lax.axis_index("s")
    wid = cid * NS + sid
    for g in range(128 // 16):
        ones_v[pl.ds(g * 16, 16)] = jnp.ones((16,), jnp.float32)

    def zb(i, _):
        zro_v[pl.ds(i * 16, 16)] = jnp.zeros((16,), jnp.float32)
        return 0
    lax.fori_loop(0, DEG_ROWS // NS // 16, zb, 0)
    pltpu.sync_copy(zro_v, deg_s.at[pl.ds(sid * (DEG_ROWS // NS), DEG_ROWS // NS)])
    plsc.subcore_barrier()
    pltpu.sync_copy(dst_hbm.at[wid], dst_v)

    def step(j, _):
        pltpu.sync_copy(ones_v, deg_s.at[dst_v.at[j]], add=True)
        return 0
    lax.fori_loop(0, EPT_A, step, 0)
    plsc.subcore_barrier()
    st = NPAD // NS
    pltpu.sync_copy(deg_s.at[pl.ds(sid * st, st)], zro_v.at[pl.ds(0, st)])
    pltpu.sync_copy(zro_v.at[pl.ds(0, st)],
                    deg_hbm.at[pl.ds(cid * NPAD + sid * st, st)])


@functools.partial(
    pl.kernel,
    out_type=jax.ShapeDtypeStruct((NC * NPAD,), jnp.float32),
    mesh=plsc.VectorSubcoreMesh(core_axis_name="c", subcore_axis_name="s"),
    compiler_params=pltpu.CompilerParams(use_tc_tiling_on_sc=False),
    scratch_types=[
        pltpu.VMEM_SHARED((DEG_ROWS,), jnp.float32),
        pltpu.VMEM((EPT_A, 128), jnp.int32),
        pltpu.VMEM((128,), jnp.float32),
        pltpu.VMEM((DEG_ROWS // NS,), jnp.float32),
    ],
)
def _deg_kernel(dst_hbm, deg_hbm, deg_s, dst_v, ones_v, zro_v):
    _deg_body(dst_hbm, deg_hbm, deg_s, dst_v, ones_v, zro_v)


def _dis_body(d0_ref, d1_ref, dis_ref):
    deg = d0_ref[...] + d1_ref[...] + 1.0
    dis_ref[...] = lax.rsqrt(deg)


def _xs_body(x_ref, dis_ref, xsa_ref, xsb_ref):
    xsa_ref[...] = x_ref[:, :D_A] * dis_ref[...]
    xsb_ref[:, :D_MOL - D_A] = x_ref[:, D_A:] * dis_ref[...]
    xsb_ref[:, D_MOL - D_A:] = jnp.zeros(
        (x_ref.shape[0], D_B - (D_MOL - D_A)), jnp.float32)


NBLK = EPT_C // (2 * CBLK)


def _make_bag_kernel(depth):
    def body(src_hbm, dst_hbm, xs_hbm, bag_hbm, bag_s, src_v, dst_v,
             r00, r01, r10, r11, i00, i01, i10, i11, sg0, sg1, ss0, ss1):
        cid = lax.axis_index("c")
        sid = lax.axis_index("s")
        base_node = cid * HALF
        rows = ((r00, r01), (r10, r11))
        idxs = ((i00, i01), (i10, i11))
        sgs = (sg0, sg1)
        sss = (ss0, ss1)
        _zero_vmem_2d(r00, 128, depth)
        zpt = BAG_ROWS // NS
        for k in range(zpt // 128):
            pltpu.sync_copy(r00,
                            bag_s.at[pl.ds(sid * zpt + k * 128, 128)])
        for k in range(zpt // 128 * 128, zpt, 64):
            pltpu.sync_copy(r00.at[pl.ds(0, 64)],
                            bag_s.at[pl.ds(sid * zpt + k, 64)])
        plsc.subcore_barrier()

        pltpu.sync_copy(src_hbm.at[sid, pl.ds(0, 2 * CBLK)], src_v)
        pltpu.sync_copy(dst_hbm.at[sid, pl.ds(0, 2 * CBLK)], dst_v)
        pltpu.async_copy(xs_hbm.at[src_v.at[0]], r00, sg0)
        pltpu.async_copy(xs_hbm.at[src_v.at[1]], r01, sg0)

        def block(sb, _):
            gather_d = None
            scat_d = None
            for jj in range(CBLK):
                s = jj % 2
                o = 1 - s
                if gather_d is None:
                    pltpu.make_async_copy(
                        xs_hbm.at[src_v.at[0]], r00, sg0).wait()
                    pltpu.make_async_copy(
                        xs_hbm.at[src_v.at[1]], r01, sg0).wait()
                else:
                    gather_d[0].wait()
                    gather_d[1].wait()
                for u in (0, 1):
                    for g in range(8):
                        v = dst_v[2 * jj + u, pl.ds(g * 16, 16)]
                        lv = v - base_node
                        ok = (lv >= 0) & (lv < HALF)
                        idxs[s][u][pl.ds(g * 16, 16)] = jnp.where(
                            ok, lv, HALF)
                new_scat = (
                    pltpu.async_copy(rows[s][0], bag_s.at[idxs[s][0]],
                                     sss[s], add=True),
                    pltpu.async_copy(rows[s][1], bag_s.at[idxs[s][1]],
                                     sss[s], add=True))
                if scat_d is None:
                    @pl.when(sb > 0)
                    def _():
                        pltpu.make_async_copy(
                            r10, bag_s.at[i10], ss1).wait()
                        pltpu.make_async_copy(
                            r11, bag_s.at[i11], ss1).wait()
                else:
                    scat_d[0].wait()
                    scat_d[1].wait()
                scat_d = new_scat
                if jj + 1 < CBLK:
                    gather_d = (
                        pltpu.async_copy(xs_hbm.at[src_v.at[2 * (jj + 1)]],
                                         rows[o][0], sgs[o]),
                        pltpu.async_copy(
                            xs_hbm.at[src_v.at[2 * (jj + 1) + 1]],
                            rows[o][1], sgs[o]))
                else:
                    gather_d = None

                    @pl.when(sb < NBLK - 1)
                    def _():
                        pltpu.sync_copy(
                            src_hbm.at[sid,
                                       pl.ds((sb + 1) * 2 * CBLK, 2 * CBLK)],
                            src_v)
                        pltpu.sync_copy(
                            dst_hbm.at[sid,
                                       pl.ds((sb + 1) * 2 * CBLK, 2 * CBLK)],
                            dst_v)
                        pltpu.async_copy(xs_hbm.at[src_v.at[0]], r00, sg0)
                        pltpu.async_copy(xs_hbm.at[src_v.at[1]], r01, sg0)
            return 0
        lax.fori_loop(0, NBLK, block, 0)
        pltpu.make_async_copy(r10, bag_s.at[i10], ss1).wait()
        pltpu.make_async_copy(r11, bag_s.at[i11], ss1).wait()
        plsc.subcore_barrier()
        st = HALF // NS
        off = 0
        while off < st:
            n = min(128, st - off)
            pltpu.sync_copy(bag_s.at[pl.ds(sid * st + off, n)],
                            r00.at[pl.ds(0, n)])
            pltpu.sync_copy(r00.at[pl.ds(0, n)],
                            bag_hbm.at[pl.ds(base_node + sid * st + off, n)])
            off += n

    return pl.kernel(
        body,
        out_type=jax.ShapeDtypeStruct((NPAD, depth), jnp.float32),
        mesh=plsc.VectorSubcoreMesh(core_axis_name="c", subcore_axis_name="s"),
        compiler_params=pltpu.CompilerParams(use_tc_tiling_on_sc=False),
        scratch_types=[
            pltpu.VMEM_SHARED((BAG_ROWS, depth), jnp.float32),
            pltpu.VMEM((2 * CBLK, 128), jnp.int32),
            pltpu.VMEM((2 * CBLK, 128), jnp.int32),
            pltpu.VMEM((128, depth), jnp.float32),
            pltpu.VMEM((128, depth), jnp.float32),
            pltpu.VMEM((128, depth), jnp.float32),
            pltpu.VMEM((128, depth), jnp.float32),
            pltpu.VMEM((128,), jnp.int32),
            pltpu.VMEM((128,), jnp.int32),
            pltpu.VMEM((128,), jnp.int32),
            pltpu.VMEM((128,), jnp.int32),
            pltpu.SemaphoreType.DMA,
            pltpu.SemaphoreType.DMA,
            pltpu.SemaphoreType.DMA,
            pltpu.SemaphoreType.DMA,
        ],
    )


_bag_kernel_a = _make_bag_kernel(D_A)
_bag_kernel_b = _make_bag_kernel(D_B)


def _h_body(baga_ref, xsa_ref, bagb_ref, xsb_ref, dis_ref, wa_ref, wb_ref,
            b_ref, h_ref):
    ta = dis_ref[...] * (baga_ref[...] + xsa_ref[...])
    tb = dis_ref[...] * (bagb_ref[...] + xsb_ref[...])
    h = (jnp.dot(ta, wa_ref[...], preferred_element_type=jnp.float32)
         + jnp.dot(tb, wb_ref[...], preferred_element_type=jnp.float32))
    h_ref[...] = jnp.maximum(h + b_ref[...], 0.0)


def _segmax_body(h_hbm, batch_hbm, parts_hbm, out_v, hv0, hv1, bv0, bv1,
                 sh0, sh1, sb0, sb1):
    cid = lax.axis_index("c")
    sid = lax.axis_index("s")
    wid = cid * NS + sid
    base = wid * RPT
    neg = jnp.full((16,), -jnp.inf, jnp.float32)
    hvs = (hv0, hv1)
    bvs = (bv0, bv1)
    shs = (sh0, sh1)
    sbs = (sb0, sb1)

    def init(i, _):
        for g in range(8):
            out_v[i, pl.ds(g * 16, 16)] = neg
        return 0
    lax.fori_loop(0, SEG_ROWS, init, 0)
    nchunk = RPT // 112
    hd = pltpu.async_copy(h_hbm.at[pl.ds(base, 112)], hv0, sh0)
    bd = pltpu.async_copy(batch_hbm.at[pl.ds(base, 112)],
                          bv0.at[pl.ds(0, 112)], sb0)
    descs = (hd, bd)
    for k in range(nchunk):
        p = k % 2
        o = 1 - p
        descs[0].wait()
        descs[1].wait()
        if k + 1 < nchunk:
            off = base + (k + 1) * 112
            hd = pltpu.async_copy(h_hbm.at[pl.ds(off, 112)], hvs[o], shs[o])
            bd = pltpu.async_copy(batch_hbm.at[pl.ds(off, 112)],
                                  bvs[o].at[pl.ds(0, 112)], sbs[o])
            descs = (hd, bd)
        hv = hvs[p]
        bv = bvs[p]

        def row(r, _):
            seg = bv[pl.ds(r, 16)][0]
            for g in range(8):
                cur = out_v[seg, pl.ds(g * 16, 16)]
                hval = hv[r, pl.ds(g * 16, 16)]
                out_v[seg, pl.ds(g * 16, 16)] = jnp.maximum(cur, hval)
            return 0
        lax.fori_loop(0, 112, row, 0)
    pltpu.sync_copy(out_v, parts_hbm.at[wid])


@functools.partial(
    pl.kernel,
    out_type=jax.ShapeDtypeStruct((NW, SEG_ROWS, D_OUT), jnp.float32),
    mesh=plsc.VectorSubcoreMesh(core_axis_name="c", subcore_axis_name="s"),
    compiler_params=pltpu.CompilerParams(use_tc_tiling_on_sc=False),
    scratch_types=[
        pltpu.VMEM((SEG_ROWS, D_OUT), jnp.float32),
        pltpu.VMEM((112, D_OUT), jnp.float32),
        pltpu.VMEM((112, D_OUT), jnp.float32),
        pltpu.VMEM((128,), jnp.int32),
        pltpu.VMEM((128,), jnp.int32),
        pltpu.SemaphoreType.DMA,
        pltpu.SemaphoreType.DMA,
        pltpu.SemaphoreType.DMA,
        pltpu.SemaphoreType.DMA,
    ],
)
def _segmax_kernel(h_hbm, batch_hbm, parts_hbm, out_v, hv0, hv1, bv0, bv1,
                   sh0, sh1, sb0, sb1):
    _segmax_body(h_hbm, batch_hbm, parts_hbm, out_v, hv0, hv1, bv0, bv1,
                 sh0, sh1, sb0, sb1)


def _head_body(parts_ref, sq_ref, w1_ref, b1_ref, w2_ref, b2_ref,
               w3_ref, b3_ref, y_ref):
    pooled = jnp.max(parts_ref[:, :B, :], axis=0)
    x2 = jnp.maximum(jnp.dot(sq_ref[...], w1_ref[...],
                             preferred_element_type=jnp.float32)
                     + b1_ref[...], 0.0)
    x2 = jnp.maximum(jnp.dot(x2, w2_ref[...],
                             preferred_element_type=jnp.float32)
                     + b2_ref[...], 0.0)
    x2 = jnp.dot(x2, w3_ref[...],
                 preferred_element_type=jnp.float32) + b3_ref[...]
    y = lax.dot_general(pooled, x2, (((1,), (1,)), ((), ())),
                        preferred_element_type=jnp.float32)
    y_ref[...] = jax.nn.sigmoid(y)


def kernel(mol_x, mol_edge_index, mol_batch, squence,
           W_gcn, b_gcn, W1, b1, W2, b2, W3, b3):
    i32 = jnp.int32
    src = mol_edge_index[0].astype(i32)
    dst = mol_edge_index[1].astype(i32)
    epad = EPAD - N_EDGES
    src_p = jnp.concatenate([src, jnp.zeros((epad,), i32)])
    dst_p = jnp.concatenate([dst, jnp.full((epad,), NPAD, i32)])
    x_p = jnp.pad(mol_x, ((0, NPAD - N_NODES), (0, 0)))
    batch_p = jnp.concatenate(
        [mol_batch.astype(i32), jnp.full((NPAD - N_NODES,), B, i32)])
    w_p = jnp.pad(W_gcn, ((0, D_A + D_B - D_MOL), (0, 0)))

    deg_parts = _deg_kernel(dst_p.reshape(NW, EPT_A, 128))

    nblk = NPAD // 1024
    dis_flat = pl.pallas_call(
        _dis_body,
        out_shape=jax.ShapeDtypeStruct((NPAD // 128, 128), jnp.float32),
        grid=(nblk,),
        in_specs=[pl.BlockSpec((8, 128), lambda i: (i, 0)),
                  pl.BlockSpec((8, 128), lambda i: (i, 0))],
        out_specs=pl.BlockSpec((8, 128), lambda i: (i, 0)),
    )(deg_parts[:NPAD].reshape(NPAD // 128, 128),
      deg_parts[NPAD:].reshape(NPAD // 128, 128))
    dis_col = dis_flat.reshape(NPAD, 1)

    xs_a, xs_b = pl.pallas_call(
        _xs_body,
        out_shape=(jax.ShapeDtypeStruct((NPAD, D_A), jnp.float32),
                   jax.ShapeDtypeStruct((NPAD, D_B), jnp.float32)),
        grid=(nblk,),
        in_specs=[pl.BlockSpec((1024, D_MOL), lambda i: (i, 0)),
                  pl.BlockSpec((1024, 1), lambda i: (i, 0))],
        out_specs=(pl.BlockSpec((1024, D_A), lambda i: (i, 0)),
                   pl.BlockSpec((1024, D_B), lambda i: (i, 0))),
    )(x_p, dis_col)

    src_r = src_p.reshape(NS, EPT_C, 128)
    dst_r = dst_p.reshape(NS, EPT_C, 128)
    bag_a = _bag_kernel_a(src_r, dst_r, xs_a)
    bag_b = _bag_kernel_b(src_r, dst_r, xs_b)

    h = pl.pallas_call(
        _h_body,
        out_shape=jax.ShapeDtypeStruct((NPAD, D_OUT), jnp.float32),
        grid=(nblk,),
        in_specs=[pl.BlockSpec((1024, D_A), lambda i: (i, 0)),
                  pl.BlockSpec((1024, D_A), lambda i: (i, 0)),
                  pl.BlockSpec((1024, D_B), lambda i: (i, 0)),
                  pl.BlockSpec((1024, D_B), lambda i: (i, 0)),
                  pl.BlockSpec((1024, 1), lambda i: (i, 0)),
                  pl.BlockSpec((D_A, D_OUT), lambda i: (0, 0)),
                  pl.BlockSpec((D_B, D_OUT), lambda i: (0, 0)),
                  pl.BlockSpec((1, D_OUT), lambda i: (0, 0))],
        out_specs=pl.BlockSpec((1024, D_OUT), lambda i: (i, 0)),
    )(bag_a, xs_a, bag_b, xs_b, dis_col, w_p[:D_A], w_p[D_A:],
      b_gcn.reshape(1, D_OUT))

    parts = _segmax_kernel(h, batch_p)

    y = pl.pallas_call(
        _head_body,
        out_shape=jax.ShapeDtypeStruct((B, B), jnp.float32),
    )(parts, squence, W1, b1.reshape(1, -1), W2, b2.reshape(1, -1),
      W3, b3.reshape(1, -1))
    return y
